# baseline (speedup 1.0000x reference)
import functools

import jax
import jax.numpy as jnp
from jax import lax
from jax.experimental import pallas as pl
from jax.experimental.pallas import tpu as pltpu
from jax.experimental.pallas import tpu_sc as plsc

_N_NODES = 10000
_N_EDGES = 160000
_EDGE_DIM = 16
_HI_B = 41
_FEAT = 32
_EHID = 32
_DP = 48
_DPB = 64
_KN = _EHID * _FEAT
_NC = 2
_NS = 16
_NW = _NC * _NS
_EPT = _N_EDGES // _NW
_GC = 1000
_NCH = _EPT // _GC
_RPT = _N_NODES // _NS

_BE = 3200
_BN = 2000


def _gather_body(x1_hbm, src_hbm, out_hbm, idx_v, rows_v, sem):
    cid = lax.axis_index("c")
    sid = lax.axis_index("s")
    wid = sid * _NC + cid
    for j in range(_NCH):
        base = wid * _EPT + j * _GC
        pltpu.sync_copy(src_hbm.at[pl.ds(base, _GC)], idx_v)
        pltpu.async_copy(x1_hbm.at[idx_v], rows_v, sem).wait()
        pltpu.sync_copy(rows_v, out_hbm.at[pl.ds(base, _GC)])


def _sc_gather(x1b, src):
    mesh = plsc.VectorSubcoreMesh(core_axis_name="c", subcore_axis_name="s")
    run = pl.kernel(
        _gather_body,
        out_type=jax.ShapeDtypeStruct((_N_EDGES, _DPB), jnp.bfloat16),
        mesh=mesh,
        scratch_types=[
            pltpu.VMEM((_GC,), jnp.int32),
            pltpu.VMEM((_GC, _DPB), jnp.bfloat16),
            pltpu.SemaphoreType.DMA,
        ],
        compiler_params=pltpu.CompilerParams(use_tc_tiling_on_sc=False),
    )
    return run(x1b, src)


def _scatter_body(msg_hbm, dst_hbm, zero_hbm, out_hbm, idx_v, rows_v, table_sh):
    cid = lax.axis_index("c")
    sid = lax.axis_index("s")
    wid = sid * _NC + cid
    pltpu.sync_copy(zero_hbm.at[pl.ds(sid * _RPT, _RPT)],
                    table_sh.at[pl.ds(sid * _RPT, _RPT)])
    plsc.subcore_barrier()
    for j in range(_NCH):
        base = wid * _EPT + j * _GC
        pltpu.sync_copy(dst_hbm.at[pl.ds(base, _GC)], idx_v)
        pltpu.sync_copy(msg_hbm.at[pl.ds(base, _GC)], rows_v)
        pltpu.sync_copy(rows_v, table_sh.at[idx_v], add=True)
    plsc.subcore_barrier()
    pltpu.sync_copy(table_sh.at[pl.ds(sid * _RPT, _RPT)],
                    out_hbm.at[cid, pl.ds(sid * _RPT, _RPT)])


def _sc_scatter(msg, dst, zeros):
    mesh = plsc.VectorSubcoreMesh(core_axis_name="c", subcore_axis_name="s")
    run = pl.kernel(
        _scatter_body,
        out_type=jax.ShapeDtypeStruct((_NC, _N_NODES, _DP), jnp.float32),
        mesh=mesh,
        scratch_types=[
            pltpu.VMEM((_GC,), jnp.int32),
            pltpu.VMEM((_GC, _DP), jnp.float32),
            pltpu.VMEM_SHARED((_N_NODES, _DP), jnp.float32),
        ],
        compiler_params=pltpu.CompilerParams(use_tc_tiling_on_sc=False),
    )
    return run(msg, dst, zeros)


def _edge_body(ea_ref, xs_ref, w1_ref, b1t_ref, w2_ref, out_ref):
    h_t = jnp.maximum(
        lax.dot_general(w1_ref[...], ea_ref[...], (((0,), (1,)), ((), ())),
                        preferred_element_type=jnp.float32)
        + b1t_ref[...], 0.0)
    G_t = lax.dot_general(w2_ref[...], xs_ref[...], (((0,), (1,)), ((), ())),
                          preferred_element_type=jnp.float32)
    acc = G_t[_KN:_KN + _FEAT, :]
    for k in range(_EHID):
        acc = acc + h_t[k:k + 1, :] * G_t[k * _FEAT:(k + 1) * _FEAT, :]
    out_ref[:, :_FEAT] = acc.T
    ones_col = (lax.broadcasted_iota(jnp.int32, (_BE, _DP - _FEAT), 1) == 0
                ).astype(jnp.float32)
    out_ref[:, _FEAT:] = ones_col


def _tc_edges(ea, xs, w1, b1t, w2aug):
    grid = _N_EDGES // _BE
    return pl.pallas_call(
        _edge_body,
        grid=(grid,),
        in_specs=[
            pl.BlockSpec((_BE, _EDGE_DIM), lambda i: (i, 0)),
            pl.BlockSpec((_BE, _DPB), lambda i: (i, 0)),
            pl.BlockSpec((_EDGE_DIM, _EHID), lambda i: (0, 0)),
            pl.BlockSpec((_EHID, 1), lambda i: (0, 0)),
            pl.BlockSpec((_DPB, _KN + _FEAT), lambda i: (0, 0)),
        ],
        out_specs=pl.BlockSpec((_BE, _DP), lambda i: (i, 0)),
        out_shape=jax.ShapeDtypeStruct((_N_EDGES, _DP), jnp.float32),
    )(ea, xs, w1, b1t, w2aug)


def _node_body(x1_ref, p_ref, rw_ref, b_ref, out_ref):
    s = p_ref[0] + p_ref[1]
    agg = s[:, :_FEAT] / jnp.maximum(s[:, _FEAT:_FEAT + 1], 1.0)
    out = (jnp.dot(x1_ref[...], rw_ref[...], preferred_element_type=jnp.float32)
           + agg + b_ref[...])
    out_ref[...] = jnp.maximum(out, 0.0)


def _tc_nodes(x1p, parts, rootp, bias2):
    grid = _N_NODES // _BN
    return pl.pallas_call(
        _node_body,
        grid=(grid,),
        in_specs=[
            pl.BlockSpec((_BN, _DP), lambda i: (i, 0)),
            pl.BlockSpec((_NC, _BN, _DP), lambda i: (0, i, 0)),
            pl.BlockSpec((_DP, _FEAT), lambda i: (0, 0)),
            pl.BlockSpec((1, _FEAT), lambda i: (0, 0)),
        ],
        out_specs=pl.BlockSpec((_BN, _FEAT), lambda i: (i, 0)),
        out_shape=jax.ShapeDtypeStruct((_N_NODES, _FEAT), jnp.float32),
    )(x1p, parts, rootp, bias2)


@jax.jit
def kernel(x, edge_index, edge_attr, lin1_w, lin1_b, lin2_w, lin2_b,
           root_w, bias):
    x1p = jnp.pad(x[:, :_HI_B], ((0, 0), (0, _DP - _HI_B)))
    x1b = jnp.pad(x[:, :_HI_B], ((0, 0), (0, _DPB - _HI_B))).astype(jnp.bfloat16)
    src = edge_index[0]
    dst = edge_index[1]
    w2r = jnp.pad(
        lin2_w.reshape(_EHID, _HI_B, _FEAT).transpose(1, 0, 2)
        .reshape(_HI_B, _KN), ((0, _DPB - _HI_B), (0, 0)))
    b2p = jnp.pad(lin2_b.reshape(_HI_B, _FEAT), ((0, _DPB - _HI_B), (0, 0)))
    w2aug = jnp.concatenate([w2r, b2p], axis=1).astype(jnp.bfloat16)
    rootp = jnp.pad(root_w, ((0, _DP - _HI_B), (0, 0)))

    x_src = _sc_gather(x1b, src)
    msg = _tc_edges(edge_attr, x_src, lin1_w, lin1_b.reshape(-1, 1), w2aug)
    parts = _sc_scatter(msg, dst, jnp.zeros((_N_NODES, _DP), jnp.float32))
    return _tc_nodes(x1p, parts, rootp, bias.reshape(1, -1))


# speedup vs baseline: 1.0821x; 1.0821x over previous
import functools

import jax
import jax.numpy as jnp
from jax import lax
from jax.experimental import pallas as pl
from jax.experimental.pallas import tpu as pltpu
from jax.experimental.pallas import tpu_sc as plsc

_N_NODES = 10000
_N_EDGES = 160000
_EDGE_DIM = 16
_HI_B = 41
_FEAT = 32
_EHID = 32
_DP = 48
_KN = _EHID * _FEAT
_NC = 2
_NS = 16
_NW = _NC * _NS
_EH = _N_EDGES // 2
_GC = 800
_NCH = _EH // _GC
_RPT = _N_NODES // _NS

_BE = 3200
_BN = 2000


def _make_gather_body(half):
    def body(x1_hbm, src_hbm, out_hbm, idx_v, rows_v, sem):
        cid = lax.axis_index("c")
        sid = lax.axis_index("s")
        wid = sid * _NC + cid

        def chunk(c):
            base = c * _GC
            pltpu.sync_copy(src_hbm.at[pl.ds(half * _EH + base, _GC)], idx_v)
            pltpu.async_copy(x1_hbm.at[idx_v], rows_v, sem).wait()
            pltpu.sync_copy(rows_v, out_hbm.at[pl.ds(base, _GC)])

        for j in range(_NCH // _NW):
            chunk(wid + _NW * j)
        c_tail = wid + _NW * (_NCH // _NW)
        if _NCH % _NW:
            @pl.when(c_tail < _NCH)
            def _():
                chunk(c_tail)
    return body


def _sc_gather(x1p, src, half):
    mesh = plsc.VectorSubcoreMesh(core_axis_name="c", subcore_axis_name="s")
    run = pl.kernel(
        _make_gather_body(half),
        out_type=jax.ShapeDtypeStruct((_EH, _DP), jnp.float32),
        mesh=mesh,
        scratch_types=[
            pltpu.VMEM((_GC,), jnp.int32),
            pltpu.VMEM((_GC, _DP), jnp.float32),
            pltpu.SemaphoreType.DMA,
        ],
        compiler_params=pltpu.CompilerParams(use_tc_tiling_on_sc=False),
    )
    return run(x1p, src)


def _make_scatter_body(half):
    def body(msg_hbm, dst_hbm, zero_hbm, out_hbm, idx_v, rows_v, table_sh):
        cid = lax.axis_index("c")
        sid = lax.axis_index("s")
        wid = sid * _NC + cid
        pltpu.sync_copy(zero_hbm.at[pl.ds(sid * _RPT, _RPT)],
                        table_sh.at[pl.ds(sid * _RPT, _RPT)])
        plsc.subcore_barrier()

        def chunk(c):
            base = c * _GC
            pltpu.sync_copy(dst_hbm.at[pl.ds(half * _EH + base, _GC)], idx_v)
            pltpu.sync_copy(msg_hbm.at[pl.ds(base, _GC)], rows_v)
            pltpu.sync_copy(rows_v, table_sh.at[idx_v], add=True)

        for j in range(_NCH // _NW):
            chunk(wid + _NW * j)
        c_tail = wid + _NW * (_NCH // _NW)
        if _NCH % _NW:
            @pl.when(c_tail < _NCH)
            def _():
                chunk(c_tail)
        plsc.subcore_barrier()
        pltpu.sync_copy(table_sh.at[pl.ds(sid * _RPT, _RPT)],
                        out_hbm.at[cid, pl.ds(sid * _RPT, _RPT)])
    return body


def _sc_scatter(msg, dst, zeros, half):
    mesh = plsc.VectorSubcoreMesh(core_axis_name="c", subcore_axis_name="s")
    run = pl.kernel(
        _make_scatter_body(half),
        out_type=jax.ShapeDtypeStruct((_NC, _N_NODES, _DP), jnp.float32),
        mesh=mesh,
        scratch_types=[
            pltpu.VMEM((_GC,), jnp.int32),
            pltpu.VMEM((_GC, _DP), jnp.float32),
            pltpu.VMEM_SHARED((_N_NODES, _DP), jnp.float32),
        ],
        compiler_params=pltpu.CompilerParams(use_tc_tiling_on_sc=False),
    )
    return run(msg, dst, zeros)


def _edge_body(ea_ref, xs_ref, w1_ref, b1t_ref, w2_ref, out_ref):
    h_t = jnp.maximum(
        lax.dot_general(w1_ref[...], ea_ref[...], (((0,), (1,)), ((), ())),
                        preferred_element_type=jnp.float32)
        + b1t_ref[...], 0.0)
    G_t = lax.dot_general(w2_ref[...], xs_ref[...], (((0,), (1,)), ((), ())),
                          preferred_element_type=jnp.float32)
    acc = G_t[_KN:_KN + _FEAT, :]
    for k in range(_EHID):
        acc = acc + h_t[k:k + 1, :] * G_t[k * _FEAT:(k + 1) * _FEAT, :]
    out_ref[:, :_FEAT] = acc.T
    ones_col = (lax.broadcasted_iota(jnp.int32, (_BE, _DP - _FEAT), 1) == 0
                ).astype(jnp.float32)
    out_ref[:, _FEAT:] = ones_col


def _tc_edges(ea, xs, w1, b1t, w2aug, half):
    grid = _EH // _BE
    off = half * grid
    return pl.pallas_call(
        _edge_body,
        grid=(grid,),
        in_specs=[
            pl.BlockSpec((_BE, _EDGE_DIM), lambda i: (i + off, 0)),
            pl.BlockSpec((_BE, _DP), lambda i: (i, 0)),
            pl.BlockSpec((_EDGE_DIM, _EHID), lambda i: (0, 0)),
            pl.BlockSpec((_EHID, 1), lambda i: (0, 0)),
            pl.BlockSpec((_DP, _KN + _FEAT), lambda i: (0, 0)),
        ],
        out_specs=pl.BlockSpec((_BE, _DP), lambda i: (i, 0)),
        out_shape=jax.ShapeDtypeStruct((_EH, _DP), jnp.float32),
    )(ea, xs, w1, b1t, w2aug)


def _node_body(x1_ref, p_ref, q_ref, rw_ref, b_ref, out_ref):
    s = (p_ref[0] + p_ref[1]) + (q_ref[0] + q_ref[1])
    agg = s[:, :_FEAT] / jnp.maximum(s[:, _FEAT:_FEAT + 1], 1.0)
    out = (jnp.dot(x1_ref[...], rw_ref[...], preferred_element_type=jnp.float32)
           + agg + b_ref[...])
    out_ref[...] = jnp.maximum(out, 0.0)


def _tc_nodes(x1p, parts0, parts1, rootp, bias2):
    grid = _N_NODES // _BN
    return pl.pallas_call(
        _node_body,
        grid=(grid,),
        in_specs=[
            pl.BlockSpec((_BN, _DP), lambda i: (i, 0)),
            pl.BlockSpec((_NC, _BN, _DP), lambda i: (0, i, 0)),
            pl.BlockSpec((_NC, _BN, _DP), lambda i: (0, i, 0)),
            pl.BlockSpec((_DP, _FEAT), lambda i: (0, 0)),
            pl.BlockSpec((1, _FEAT), lambda i: (0, 0)),
        ],
        out_specs=pl.BlockSpec((_BN, _FEAT), lambda i: (i, 0)),
        out_shape=jax.ShapeDtypeStruct((_N_NODES, _FEAT), jnp.float32),
    )(x1p, parts0, parts1, rootp, bias2)


@jax.jit
def kernel(x, edge_index, edge_attr, lin1_w, lin1_b, lin2_w, lin2_b,
           root_w, bias):
    x1p = jnp.pad(x[:, :_HI_B], ((0, 0), (0, _DP - _HI_B)))
    src = edge_index[0]
    dst = edge_index[1]
    w2r = jnp.pad(
        lin2_w.reshape(_EHID, _HI_B, _FEAT).transpose(1, 0, 2)
        .reshape(_HI_B, _KN), ((0, _DP - _HI_B), (0, 0)))
    b2p = jnp.pad(lin2_b.reshape(_HI_B, _FEAT), ((0, _DP - _HI_B), (0, 0)))
    w2aug = jnp.concatenate([w2r, b2p], axis=1)
    rootp = jnp.pad(root_w, ((0, _DP - _HI_B), (0, 0)))

    zeros = jnp.zeros((_N_NODES, _DP), jnp.float32)
    b1t = lin1_b.reshape(-1, 1)
    xs0 = _sc_gather(x1p, src, 0)
    xs1 = _sc_gather(x1p, src, 1)
    msg0 = _tc_edges(edge_attr, xs0, lin1_w, b1t, w2aug, 0)
    msg1 = _tc_edges(edge_attr, xs1, lin1_w, b1t, w2aug, 1)
    parts0 = _sc_scatter(msg0, dst, zeros, 0)
    parts1 = _sc_scatter(msg1, dst, zeros, 1)
    return _tc_nodes(x1p, parts0, parts1, rootp, bias.reshape(1, -1))
